# baseline (speedup 1.0000x reference)
import functools

import jax
import jax.numpy as jnp
from jax import lax
from jax.experimental import pallas as pl
from jax.experimental.pallas import tpu as pltpu
from jax.experimental.pallas import tpu_sc as plsc

B, T, V, H = 1024, 50, 1000, 1024
L = 16
TP = 64
VP = 1024


def _make_counts_kernel():
    info = plsc.get_sparse_core_info()
    nc, ns = info.num_cores, info.num_subcores
    nw = nc * ns
    bpw = B // nw

    mesh = plsc.VectorSubcoreMesh(core_axis_name="c", subcore_axis_name="s")

    @functools.partial(
        pl.kernel,
        mesh=mesh,
        compiler_params=pltpu.CompilerParams(
            needs_layout_passes=False,
            use_tc_tiling_on_sc=False,
        ),
        out_type=jax.ShapeDtypeStruct((nw, bpw * VP), jnp.float32),
        scratch_types=[
            pltpu.VMEM((bpw * TP,), jnp.int32),
            pltpu.VMEM((bpw * VP,), jnp.float32),
        ],
    )
    def k(idx_hbm, cnt_hbm, idx_v, cnt_v):
        wid = lax.axis_index("s") * nc + lax.axis_index("c")
        pltpu.sync_copy(idx_hbm.at[wid], idx_v)

        zeros = jnp.zeros((L,), jnp.float32)

        def zloop(i, _):
            for u in range(8):
                cnt_v[pl.ds((i * 8 + u) * L, L)] = zeros
            return 0

        lax.fori_loop(0, bpw * VP // (8 * L), zloop, 0)

        ones = jnp.ones((L,), jnp.float32)
        for r in range(bpw):
            rbase = r * VP
            for g in range(TP // L):
                addr = idx_v[pl.ds(r * TP + g * L, L)] + rbase
                plsc.addupdate_scatter(cnt_v, [addr], ones)

        pltpu.sync_copy(cnt_v, cnt_hbm.at[wid])

    return k


_counts_kernel = _make_counts_kernel()


def _mm_body(a_ref, b_ref, o_ref):
    o_ref[...] = jnp.dot(a_ref[:, :V], b_ref[...],
                         preferred_element_type=jnp.float32)


_BM = 256
_matmul = pl.pallas_call(
    _mm_body,
    grid=(B // _BM,),
    in_specs=[
        pl.BlockSpec((_BM, VP), lambda i: (i, 0)),
        pl.BlockSpec((V, H), lambda i: (0, 0)),
    ],
    out_specs=pl.BlockSpec((_BM, H), lambda i: (i, 0)),
    out_shape=jax.ShapeDtypeStruct((B, H), jnp.float32),
)


def kernel(indices, codebook):
    idx = jnp.pad(indices.astype(jnp.int32), ((0, 0), (0, TP - T)),
                  constant_values=V)
    idx = idx.reshape(B // 32, 32 * TP)
    counts = _counts_kernel(idx).reshape(B, VP).astype(jnp.bfloat16)
    return _matmul(counts, codebook.astype(jnp.bfloat16))


# speedup vs baseline: 1.0723x; 1.0723x over previous
import functools

import jax
import jax.numpy as jnp
from jax import lax
from jax.experimental import pallas as pl
from jax.experimental.pallas import tpu as pltpu
from jax.experimental.pallas import tpu_sc as plsc

B, T, V, H = 1024, 50, 1000, 1024
L = 16
VP = 1024


def _make_counts_kernel():
    info = plsc.get_sparse_core_info()
    nc, ns = info.num_cores, info.num_subcores
    nw = nc * ns
    bpw = B // nw
    nt = bpw * T
    nt_pad = nt + L

    mesh = plsc.VectorSubcoreMesh(core_axis_name="c", subcore_axis_name="s")

    @functools.partial(
        pl.kernel,
        mesh=mesh,
        compiler_params=pltpu.CompilerParams(
            needs_layout_passes=False,
            use_tc_tiling_on_sc=False,
        ),
        out_type=jax.ShapeDtypeStruct((nw, bpw * VP), jnp.float32),
        scratch_types=[
            pltpu.VMEM((nt_pad,), jnp.int32),
            pltpu.VMEM((bpw * VP,), jnp.float32),
        ],
    )
    def k(idx_hbm, cnt_hbm, idx_v, cnt_v):
        wid = lax.axis_index("s") * nc + lax.axis_index("c")
        pltpu.sync_copy(idx_hbm.at[wid], idx_v.at[pl.ds(0, nt)])

        zeros = jnp.zeros((L,), jnp.float32)

        def zloop(i, _):
            for u in range(8):
                cnt_v[pl.ds((i * 8 + u) * L, L)] = zeros
            return 0

        lax.fori_loop(0, bpw * VP // (8 * L), zloop, 0)

        ones = jnp.ones((L,), jnp.float32)
        lane = lax.iota(jnp.int32, L)
        for r in range(bpw):
            rbase = r * VP
            for g in range(-(-T // L)):
                valid = min(L, T - g * L)
                addr = idx_v[pl.ds(r * T + g * L, L)] + rbase
                if valid == L:
                    plsc.addupdate_scatter(cnt_v, [addr], ones)
                else:
                    plsc.addupdate_scatter(cnt_v, [addr], ones,
                                           mask=lane < valid)

        pltpu.sync_copy(cnt_v, cnt_hbm.at[wid])

    return k


_counts_kernel = _make_counts_kernel()


def _mm_body(a_ref, b_ref, o_ref):
    o_ref[...] = jnp.dot(a_ref[:, :V], b_ref[...],
                         preferred_element_type=jnp.float32)


_BM = 256
_matmul = pl.pallas_call(
    _mm_body,
    grid=(B // _BM,),
    in_specs=[
        pl.BlockSpec((_BM, VP), lambda i: (i, 0)),
        pl.BlockSpec((V, H), lambda i: (0, 0)),
    ],
    out_specs=pl.BlockSpec((_BM, H), lambda i: (i, 0)),
    out_shape=jax.ShapeDtypeStruct((B, H), jnp.float32),
)


def kernel(indices, codebook):
    idx = indices.astype(jnp.int32).reshape(32, B // 32 * T)
    counts = _counts_kernel(idx).reshape(B, VP)
    return _matmul(counts, codebook)
